# baseline (speedup 1.0000x reference)
import functools

import jax
import jax.numpy as jnp
from jax import lax
from jax.experimental import pallas as pl
from jax.experimental.pallas import tpu as pltpu
from jax.experimental.pallas import tpu_sc as plsc

BATCH = 4096
HIDDEN = 128
_INFO = plsc.get_sparse_core_info()
NC, NS, L = _INFO.num_cores, _INFO.num_subcores, _INFO.num_lanes
NW = NC * NS
RPW = BATCH // NW
LANE_CHUNKS = HIDDEN // L
NCHUNK = 8
RPC = RPW // NCHUNK


def _vtbpr_body(users_hbm, items_hbm, vf_hbm, tf_hbm,
                ug_hbm, ig_hbm, ubeta_hbm, ibeta_hbm, tuv_hbm, tut_hbm,
                out_hbm,
                uidx_v, iidx_v, ug_v, ig_v, tuv_v, tut_v, vf_v, tf_v,
                ub_v, ib_v, out_v, sems, bsem):
    wid = lax.axis_index("s") * NC + lax.axis_index("c")
    base = wid * RPW

    icopies = [
        pltpu.async_copy(users_hbm.at[pl.ds(base, RPW)], uidx_v, bsem),
        pltpu.async_copy(items_hbm.at[pl.ds(base, RPW)], iidx_v, bsem),
    ]
    for h in icopies:
        h.wait()

    def fire(c):
        rs = pl.ds(c * RPC, RPC)
        sem = sems.at[c]
        return [
            pltpu.async_copy(ug_hbm.at[uidx_v.at[rs]], ug_v.at[rs], sem),
            pltpu.async_copy(ig_hbm.at[iidx_v.at[rs]], ig_v.at[rs], sem),
            pltpu.async_copy(tuv_hbm.at[uidx_v.at[rs]], tuv_v.at[rs], sem),
            pltpu.async_copy(tut_hbm.at[uidx_v.at[rs]], tut_v.at[rs], sem),
            pltpu.async_copy(vf_hbm.at[pl.ds(base + c * RPC, RPC)], vf_v.at[rs], sem),
            pltpu.async_copy(tf_hbm.at[pl.ds(base + c * RPC, RPC)], tf_v.at[rs], sem),
        ]

    last_lane = lax.broadcasted_iota(jnp.int32, (L,), 0) == (L - 1)

    handles = {0: fire(0)}
    bcopies = None
    for c in range(NCHUNK):
        for h in handles.pop(c):
            h.wait()
        if c + 1 < NCHUNK:
            handles[c + 1] = fire(c + 1)
        if c == NCHUNK - 2:
            bcopies = [
                pltpu.async_copy(ubeta_hbm.at[uidx_v], ub_v, bsem),
                pltpu.async_copy(ibeta_hbm.at[iidx_v], ib_v, bsem),
            ]

        @plsc.parallel_loop(c * RPC, (c + 1) * RPC, unroll=2)
        def row(r):
            acc = ug_v[r, pl.ds(0, L)] * ig_v[r, pl.ds(0, L)]
            for j in range(LANE_CHUNKS):
                sl = pl.ds(j * L, L)
                if j:
                    acc = acc + ug_v[r, sl] * ig_v[r, sl]
                acc = acc + tuv_v[r, sl] * vf_v[r, sl]
                acc = acc + tut_v[r, sl] * tf_v[r, sl]
            tot = plsc.cumsum(acc)
            idx = jnp.full((L,), r, jnp.int32)
            plsc.store_scatter(out_v, [idx], tot, mask=last_lane)

    for h in bcopies:
        h.wait()
    for j in range(RPW // L):
        sl = pl.ds(j * L, L)
        out_v[sl] = out_v[sl] + ub_v[sl] + ib_v[sl]

    pltpu.sync_copy(out_v, out_hbm.at[pl.ds(base, RPW)])


@jax.jit
def _vtbpr(users, items, vf, tf, ug, ig, ubeta, ibeta, tuv, tut):
    mesh = plsc.VectorSubcoreMesh(core_axis_name="c", subcore_axis_name="s")
    run = functools.partial(
        pl.kernel, mesh=mesh,
        compiler_params=pltpu.CompilerParams(
            needs_layout_passes=False,
            disable_bounds_checks=True,
            disable_semaphore_checks=True,
            skip_device_barrier=True,
        ),
        out_type=jax.ShapeDtypeStruct((BATCH,), jnp.float32),
        scratch_types=[
            pltpu.VMEM((RPW,), jnp.int32),
            pltpu.VMEM((RPW,), jnp.int32),
            pltpu.VMEM((RPW, HIDDEN), jnp.float32),
            pltpu.VMEM((RPW, HIDDEN), jnp.float32),
            pltpu.VMEM((RPW, HIDDEN), jnp.float32),
            pltpu.VMEM((RPW, HIDDEN), jnp.float32),
            pltpu.VMEM((RPW, HIDDEN), jnp.float32),
            pltpu.VMEM((RPW, HIDDEN), jnp.float32),
            pltpu.VMEM((RPW,), jnp.float32),
            pltpu.VMEM((RPW,), jnp.float32),
            pltpu.VMEM((RPW,), jnp.float32),
            pltpu.SemaphoreType.DMA((NCHUNK,)),
            pltpu.SemaphoreType.DMA,
        ],
    )(_vtbpr_body)
    return run(users, items, vf, tf, ug, ig, ubeta, ibeta, tuv, tut)


def kernel(users, items, visual_features, textural_features,
           user_gama, item_gama, user_beta, item_beta,
           theta_user_visual, theta_user_text):
    return _vtbpr(users, items, visual_features, textural_features,
                  user_gama, item_gama,
                  user_beta.reshape(-1), item_beta.reshape(-1),
                  theta_user_visual, theta_user_text)


# speedup vs baseline: 1.1672x; 1.1672x over previous
import functools

import jax
import jax.numpy as jnp
from jax import lax
from jax.experimental import pallas as pl
from jax.experimental.pallas import tpu as pltpu
from jax.experimental.pallas import tpu_sc as plsc

BATCH = 4096
HIDDEN = 128
_INFO = plsc.get_sparse_core_info()
NC, NS, L = _INFO.num_cores, _INFO.num_subcores, _INFO.num_lanes
NW = NC * NS
RPW = BATCH // NW
LANE_CHUNKS = HIDDEN // L
NCHUNK = 2
RPC = RPW // NCHUNK


def _vtbpr_body(users_hbm, items_hbm, vf_hbm, tf_hbm,
                ug_hbm, ig_hbm, ubeta_hbm, ibeta_hbm, tuv_hbm, tut_hbm,
                out_hbm,
                uidx_v, iidx_v, ug_v, ig_v, tuv_v, tut_v, vf_v, tf_v,
                ub_v, ib_v, out_v, sems, bsem):
    wid = lax.axis_index("s") * NC + lax.axis_index("c")
    base = wid * RPW

    icopies = [
        pltpu.async_copy(users_hbm.at[pl.ds(base, RPW)], uidx_v, bsem),
        pltpu.async_copy(items_hbm.at[pl.ds(base, RPW)], iidx_v, bsem),
    ]
    for h in icopies:
        h.wait()

    def fire(c):
        rs = pl.ds(c * RPC, RPC)
        sem = sems.at[c]
        return [
            pltpu.async_copy(ug_hbm.at[uidx_v.at[rs]], ug_v.at[rs], sem),
            pltpu.async_copy(ig_hbm.at[iidx_v.at[rs]], ig_v.at[rs], sem),
            pltpu.async_copy(tuv_hbm.at[uidx_v.at[rs]], tuv_v.at[rs], sem),
            pltpu.async_copy(tut_hbm.at[uidx_v.at[rs]], tut_v.at[rs], sem),
            pltpu.async_copy(vf_hbm.at[pl.ds(base + c * RPC, RPC)], vf_v.at[rs], sem),
            pltpu.async_copy(tf_hbm.at[pl.ds(base + c * RPC, RPC)], tf_v.at[rs], sem),
        ]

    last_lane = lax.broadcasted_iota(jnp.int32, (L,), 0) == (L - 1)

    handles = {0: fire(0)}
    bcopies = None
    for c in range(NCHUNK):
        for h in handles.pop(c):
            h.wait()
        if c + 1 < NCHUNK:
            handles[c + 1] = fire(c + 1)
        if c == NCHUNK - 2:
            bcopies = [
                pltpu.async_copy(ubeta_hbm.at[uidx_v], ub_v, bsem),
                pltpu.async_copy(ibeta_hbm.at[iidx_v], ib_v, bsem),
            ]

        @plsc.parallel_loop(c * RPC, (c + 1) * RPC, unroll=2)
        def row(r):
            acc = ug_v[r, pl.ds(0, L)] * ig_v[r, pl.ds(0, L)]
            for j in range(LANE_CHUNKS):
                sl = pl.ds(j * L, L)
                if j:
                    acc = acc + ug_v[r, sl] * ig_v[r, sl]
                acc = acc + tuv_v[r, sl] * vf_v[r, sl]
                acc = acc + tut_v[r, sl] * tf_v[r, sl]
            tot = plsc.cumsum(acc)
            idx = jnp.full((L,), r, jnp.int32)
            plsc.store_scatter(out_v, [idx], tot, mask=last_lane)

    for h in bcopies:
        h.wait()
    for j in range(RPW // L):
        sl = pl.ds(j * L, L)
        out_v[sl] = out_v[sl] + ub_v[sl] + ib_v[sl]

    pltpu.sync_copy(out_v, out_hbm.at[pl.ds(base, RPW)])


@jax.jit
def _vtbpr(users, items, vf, tf, ug, ig, ubeta, ibeta, tuv, tut):
    mesh = plsc.VectorSubcoreMesh(core_axis_name="c", subcore_axis_name="s")
    run = functools.partial(
        pl.kernel, mesh=mesh,
        compiler_params=pltpu.CompilerParams(
            needs_layout_passes=False,
            disable_bounds_checks=True,
            disable_semaphore_checks=True,
            skip_device_barrier=True,
        ),
        out_type=jax.ShapeDtypeStruct((BATCH,), jnp.float32),
        scratch_types=[
            pltpu.VMEM((RPW,), jnp.int32),
            pltpu.VMEM((RPW,), jnp.int32),
            pltpu.VMEM((RPW, HIDDEN), jnp.float32),
            pltpu.VMEM((RPW, HIDDEN), jnp.float32),
            pltpu.VMEM((RPW, HIDDEN), jnp.float32),
            pltpu.VMEM((RPW, HIDDEN), jnp.float32),
            pltpu.VMEM((RPW, HIDDEN), jnp.float32),
            pltpu.VMEM((RPW, HIDDEN), jnp.float32),
            pltpu.VMEM((RPW,), jnp.float32),
            pltpu.VMEM((RPW,), jnp.float32),
            pltpu.VMEM((RPW,), jnp.float32),
            pltpu.SemaphoreType.DMA((NCHUNK,)),
            pltpu.SemaphoreType.DMA,
        ],
    )(_vtbpr_body)
    return run(users, items, vf, tf, ug, ig, ubeta, ibeta, tuv, tut)


def kernel(users, items, visual_features, textural_features,
           user_gama, item_gama, user_beta, item_beta,
           theta_user_visual, theta_user_text):
    return _vtbpr(users, items, visual_features, textural_features,
                  user_gama, item_gama,
                  user_beta.reshape(-1), item_beta.reshape(-1),
                  theta_user_visual, theta_user_text)


# speedup vs baseline: 1.1826x; 1.0131x over previous
import functools

import jax
import jax.numpy as jnp
from jax import lax
from jax.experimental import pallas as pl
from jax.experimental.pallas import tpu as pltpu
from jax.experimental.pallas import tpu_sc as plsc

BATCH = 4096
HIDDEN = 128
_INFO = plsc.get_sparse_core_info()
NC, NS, L = _INFO.num_cores, _INFO.num_subcores, _INFO.num_lanes
NW = NC * NS
RPW = BATCH // NW
LANE_CHUNKS = HIDDEN // L
NCHUNK = 2
RPC = RPW // NCHUNK


def _vtbpr_body(users_hbm, items_hbm, vf_hbm, tf_hbm,
                ug_hbm, ig_hbm, ubeta_hbm, ibeta_hbm, tuv_hbm, tut_hbm,
                out_hbm,
                uidx_v, iidx_v, ug_v, ig_v, tuv_v, tut_v, vf_v, tf_v,
                ub_v, ib_v, out_v, sems, bsem, fsem):
    wid = lax.axis_index("s") * NC + lax.axis_index("c")
    base = wid * RPW

    icopies = [
        pltpu.async_copy(users_hbm.at[pl.ds(base, RPW)], uidx_v, bsem),
        pltpu.async_copy(items_hbm.at[pl.ds(base, RPW)], iidx_v, bsem),
    ]
    fcopies = [
        pltpu.async_copy(vf_hbm.at[pl.ds(base, RPW)], vf_v, fsem),
        pltpu.async_copy(tf_hbm.at[pl.ds(base, RPW)], tf_v, fsem),
    ]
    for h in icopies:
        h.wait()

    def fire(c):
        rs = pl.ds(c * RPC, RPC)
        sem = sems.at[c]
        return [
            pltpu.async_copy(ug_hbm.at[uidx_v.at[rs]], ug_v.at[rs], sem),
            pltpu.async_copy(ig_hbm.at[iidx_v.at[rs]], ig_v.at[rs], sem),
            pltpu.async_copy(tuv_hbm.at[uidx_v.at[rs]], tuv_v.at[rs], sem),
            pltpu.async_copy(tut_hbm.at[uidx_v.at[rs]], tut_v.at[rs], sem),
        ]

    last_lane = lax.broadcasted_iota(jnp.int32, (L,), 0) == (L - 1)

    handles = {0: fire(0)}
    bcopies = [
        pltpu.async_copy(ubeta_hbm.at[uidx_v], ub_v, bsem),
        pltpu.async_copy(ibeta_hbm.at[iidx_v], ib_v, bsem),
    ]
    for c in range(NCHUNK):
        for h in handles.pop(c):
            h.wait()
        if c + 1 < NCHUNK:
            handles[c + 1] = fire(c + 1)
        if c == 0:
            for h in fcopies:
                h.wait()

        @plsc.parallel_loop(c * RPC, (c + 1) * RPC, unroll=2)
        def row(r):
            acc = ug_v[r, pl.ds(0, L)] * ig_v[r, pl.ds(0, L)]
            for j in range(LANE_CHUNKS):
                sl = pl.ds(j * L, L)
                if j:
                    acc = acc + ug_v[r, sl] * ig_v[r, sl]
                acc = acc + tuv_v[r, sl] * vf_v[r, sl]
                acc = acc + tut_v[r, sl] * tf_v[r, sl]
            tot = plsc.cumsum(acc)
            idx = jnp.full((L,), r, jnp.int32)
            plsc.store_scatter(out_v, [idx], tot, mask=last_lane)

    for h in bcopies:
        h.wait()
    for j in range(RPW // L):
        sl = pl.ds(j * L, L)
        out_v[sl] = out_v[sl] + ub_v[sl] + ib_v[sl]

    pltpu.sync_copy(out_v, out_hbm.at[pl.ds(base, RPW)])


@jax.jit
def _vtbpr(users, items, vf, tf, ug, ig, ubeta, ibeta, tuv, tut):
    mesh = plsc.VectorSubcoreMesh(core_axis_name="c", subcore_axis_name="s")
    run = functools.partial(
        pl.kernel, mesh=mesh,
        compiler_params=pltpu.CompilerParams(
            needs_layout_passes=False,
            disable_bounds_checks=True,
            disable_semaphore_checks=True,
            skip_device_barrier=True,
        ),
        out_type=jax.ShapeDtypeStruct((BATCH,), jnp.float32),
        scratch_types=[
            pltpu.VMEM((RPW,), jnp.int32),
            pltpu.VMEM((RPW,), jnp.int32),
            pltpu.VMEM((RPW, HIDDEN), jnp.float32),
            pltpu.VMEM((RPW, HIDDEN), jnp.float32),
            pltpu.VMEM((RPW, HIDDEN), jnp.float32),
            pltpu.VMEM((RPW, HIDDEN), jnp.float32),
            pltpu.VMEM((RPW, HIDDEN), jnp.float32),
            pltpu.VMEM((RPW, HIDDEN), jnp.float32),
            pltpu.VMEM((RPW,), jnp.float32),
            pltpu.VMEM((RPW,), jnp.float32),
            pltpu.VMEM((RPW,), jnp.float32),
            pltpu.SemaphoreType.DMA((NCHUNK,)),
            pltpu.SemaphoreType.DMA,
            pltpu.SemaphoreType.DMA,
        ],
    )(_vtbpr_body)
    return run(users, items, vf, tf, ug, ig, ubeta, ibeta, tuv, tut)


def kernel(users, items, visual_features, textural_features,
           user_gama, item_gama, user_beta, item_beta,
           theta_user_visual, theta_user_text):
    return _vtbpr(users, items, visual_features, textural_features,
                  user_gama, item_gama,
                  user_beta.reshape(-1), item_beta.reshape(-1),
                  theta_user_visual, theta_user_text)


# speedup vs baseline: 1.2422x; 1.0504x over previous
import functools

import jax
import jax.numpy as jnp
from jax import lax
from jax.experimental import pallas as pl
from jax.experimental.pallas import tpu as pltpu
from jax.experimental.pallas import tpu_sc as plsc

BATCH = 4096
HIDDEN = 128
_INFO = plsc.get_sparse_core_info()
NC, NS, L = _INFO.num_cores, _INFO.num_subcores, _INFO.num_lanes
NW = NC * NS
RPW = BATCH // NW
LANE_CHUNKS = HIDDEN // L
NCHUNK = 2
RPC = RPW // NCHUNK


NUSER = 100000
NITEM = 100000


def _vtbpr_body(users_hbm, items_hbm, vf_hbm, tf_hbm,
                ug_hbm, ig_hbm, ubeta_hbm, ibeta_hbm, tuv_hbm, tut_hbm,
                out_hbm,
                uidx_v, iidx_v, ug_v, ig_v, tuv_v, tut_v, vf_v, tf_v,
                ub_v, ib_v, out_v, ubs_s, ibs_s, sems, bsem, fsem, tsem):
    s_idx = lax.axis_index("s")
    wid = s_idx * NC + lax.axis_index("c")
    base = wid * RPW

    icopies = [
        pltpu.async_copy(users_hbm.at[pl.ds(base, RPW)], uidx_v, bsem),
        pltpu.async_copy(items_hbm.at[pl.ds(base, RPW)], iidx_v, bsem),
    ]
    fcopies = [
        pltpu.async_copy(vf_hbm.at[pl.ds(base, RPW)], vf_v, fsem),
        pltpu.async_copy(tf_hbm.at[pl.ds(base, RPW)], tf_v, fsem),
    ]
    @pl.when(s_idx == 0)
    def _():
        pltpu.async_copy(ubeta_hbm.at[0], ubs_s, tsem)
        pltpu.async_copy(ibeta_hbm.at[0], ibs_s, tsem)

    for h in icopies:
        h.wait()

    def fire(c):
        rs = pl.ds(c * RPC, RPC)
        sem = sems.at[c]
        return [
            pltpu.async_copy(ug_hbm.at[uidx_v.at[rs]], ug_v.at[rs], sem),
            pltpu.async_copy(ig_hbm.at[iidx_v.at[rs]], ig_v.at[rs], sem),
            pltpu.async_copy(tuv_hbm.at[uidx_v.at[rs]], tuv_v.at[rs], sem),
            pltpu.async_copy(tut_hbm.at[uidx_v.at[rs]], tut_v.at[rs], sem),
        ]

    last_lane = lax.broadcasted_iota(jnp.int32, (L,), 0) == (L - 1)

    handles = {0: fire(0)}
    for c in range(NCHUNK):
        for h in handles.pop(c):
            h.wait()
        if c + 1 < NCHUNK:
            handles[c + 1] = fire(c + 1)
        if c == 0:
            for h in fcopies:
                h.wait()

        @plsc.parallel_loop(c * RPC, (c + 1) * RPC, unroll=2)
        def row(r):
            acc = ug_v[r, pl.ds(0, L)] * ig_v[r, pl.ds(0, L)]
            for j in range(LANE_CHUNKS):
                sl = pl.ds(j * L, L)
                if j:
                    acc = acc + ug_v[r, sl] * ig_v[r, sl]
                acc = acc + tuv_v[r, sl] * vf_v[r, sl]
                acc = acc + tut_v[r, sl] * tf_v[r, sl]
            tot = plsc.cumsum(acc)
            idx = jnp.full((L,), r, jnp.int32)
            plsc.store_scatter(out_v, [idx], tot, mask=last_lane)

    @pl.when(s_idx == 0)
    def _():
        pltpu.make_async_copy(ubeta_hbm.at[0], ubs_s, tsem).wait()
        pltpu.make_async_copy(ibeta_hbm.at[0], ibs_s, tsem).wait()

    plsc.subcore_barrier()
    bcopies = [
        pltpu.async_copy(ubs_s.at[uidx_v], ub_v, bsem),
        pltpu.async_copy(ibs_s.at[iidx_v], ib_v, bsem),
    ]
    for h in bcopies:
        h.wait()
    for j in range(RPW // L):
        sl = pl.ds(j * L, L)
        out_v[sl] = out_v[sl] + ub_v[sl] + ib_v[sl]

    pltpu.sync_copy(out_v, out_hbm.at[pl.ds(base, RPW)])


@jax.jit
def _vtbpr(users, items, vf, tf, ug, ig, ubeta, ibeta, tuv, tut):
    mesh = plsc.VectorSubcoreMesh(core_axis_name="c", subcore_axis_name="s")
    run = functools.partial(
        pl.kernel, mesh=mesh,
        compiler_params=pltpu.CompilerParams(
            needs_layout_passes=False,
            disable_bounds_checks=True,
            disable_semaphore_checks=True,
            skip_device_barrier=True,
        ),
        out_type=jax.ShapeDtypeStruct((BATCH,), jnp.float32),
        scratch_types=[
            pltpu.VMEM((RPW,), jnp.int32),
            pltpu.VMEM((RPW,), jnp.int32),
            pltpu.VMEM((RPW, HIDDEN), jnp.float32),
            pltpu.VMEM((RPW, HIDDEN), jnp.float32),
            pltpu.VMEM((RPW, HIDDEN), jnp.float32),
            pltpu.VMEM((RPW, HIDDEN), jnp.float32),
            pltpu.VMEM((RPW, HIDDEN), jnp.float32),
            pltpu.VMEM((RPW, HIDDEN), jnp.float32),
            pltpu.VMEM((RPW,), jnp.float32),
            pltpu.VMEM((RPW,), jnp.float32),
            pltpu.VMEM((RPW,), jnp.float32),
            pltpu.VMEM_SHARED((NUSER,), jnp.float32),
            pltpu.VMEM_SHARED((NITEM,), jnp.float32),
            pltpu.SemaphoreType.DMA((NCHUNK,)),
            pltpu.SemaphoreType.DMA,
            pltpu.SemaphoreType.DMA,
            pltpu.SemaphoreType.DMA,
        ],
    )(_vtbpr_body)
    return run(users, items, vf, tf, ug, ig, ubeta, ibeta, tuv, tut)


def kernel(users, items, visual_features, textural_features,
           user_gama, item_gama, user_beta, item_beta,
           theta_user_visual, theta_user_text):
    return _vtbpr(users, items, visual_features, textural_features,
                  user_gama, item_gama,
                  user_beta.T, item_beta.T,
                  theta_user_visual, theta_user_text)
